# Initial kernel scaffold; baseline (speedup 1.0000x reference)
import jax
import jax.numpy as jnp
from jax.experimental import pallas as pl


def kernel(x, edge_index, att):
    raise NotImplementedError("write your pallas kernel here")


# speedup vs baseline: 4.9987x; 4.9987x over previous
import functools

import jax
import jax.numpy as jnp
from jax import lax
from jax.experimental import pallas as pl
from jax.experimental.pallas import tpu as pltpu
from jax.experimental.pallas import tpu_sc as plsc

F_IN = 128
N_HEADS = 4
N_NODES = 10000
N_EDGES = 320000

NUM_CORES = 2
NUM_SUBCORES = 16
NUM_WORKERS = NUM_CORES * NUM_SUBCORES
LANES = 16

TW = 2 * N_HEADS
EDGES_PER_WORKER = N_EDGES // NUM_WORKERS
SUB = 2000
N_SUB = EDGES_PER_WORKER // SUB
GROUPS = SUB // LANES


def _node_projection(x, w):
    def body(x_ref, w_ref, y_ref):
        y_ref[...] = jnp.dot(x_ref[...], w_ref[...],
                             preferred_element_type=jnp.float32)

    m_blk = 2000
    return pl.pallas_call(
        body,
        grid=(N_NODES // m_blk,),
        in_specs=[pl.BlockSpec((m_blk, F_IN), lambda i: (i, 0)),
                  pl.BlockSpec((F_IN, TW), lambda i: (0, 0))],
        out_specs=pl.BlockSpec((m_blk, TW), lambda i: (i, 0)),
        out_shape=jax.ShapeDtypeStruct((N_NODES, TW), jnp.float32),
    )(x, w)


def _make_edge_lift():
    mesh = plsc.VectorSubcoreMesh(core_axis_name="c", subcore_axis_name="s")

    @functools.partial(
        pl.kernel,
        out_type=jax.ShapeDtypeStruct((N_EDGES * N_HEADS,), jnp.float32),
        mesh=mesh,
        compiler_params=pltpu.CompilerParams(needs_layout_passes=False),
        scratch_types=[
            pltpu.VMEM((N_NODES * TW,), jnp.float32),
            pltpu.VMEM((SUB,), jnp.int32),
            pltpu.VMEM((SUB,), jnp.int32),
            pltpu.VMEM((SUB * N_HEADS,), jnp.float32),
        ],
    )
    def edge_lift(y_hbm, src_hbm, dst_hbm, out_hbm,
                  table_v, src_v, dst_v, out_v):
        wid = lax.axis_index("s") * NUM_CORES + lax.axis_index("c")
        pltpu.sync_copy(y_hbm, table_v)
        lane4 = lax.shift_left(lax.iota(jnp.int32, LANES), 2)
        base_e = wid * EDGES_PER_WORKER
        for c in range(N_SUB):
            cb = base_e + c * SUB
            pltpu.sync_copy(src_hbm.at[pl.ds(cb, SUB)], src_v)
            pltpu.sync_copy(dst_hbm.at[pl.ds(cb, SUB)], dst_v)

            def group(g, carry):
                s = src_v[pl.ds(g * LANES, LANES)]
                d = dst_v[pl.ds(g * LANES, LANES)]
                s8 = lax.shift_left(s, 3)
                d8 = lax.shift_left(d, 3) + N_HEADS
                e4 = lane4 + g * (LANES * N_HEADS)
                for k in range(N_HEADS):
                    a = plsc.load_gather(table_v, [s8 + k])
                    b = plsc.load_gather(table_v, [d8 + k])
                    v = jnp.maximum(a + b, 0.0)
                    plsc.store_scatter(out_v, [e4 + k], v)
                return carry

            lax.fori_loop(0, GROUPS, group, 0)
            pltpu.sync_copy(out_v, out_hbm.at[pl.ds(cb * N_HEADS,
                                                    SUB * N_HEADS)])

    return edge_lift


def kernel(x, edge_index, att):
    att2 = att[:, :, 0]
    w = jnp.concatenate([att2[:, :F_IN].T, att2[:, F_IN:].T], axis=1)
    y = _node_projection(x, w)
    out_flat = _make_edge_lift()(y.reshape(-1), edge_index[0], edge_index[1])
    return out_flat.reshape(N_EDGES, N_HEADS)


# speedup vs baseline: 19.1347x; 3.8279x over previous
import functools

import jax
import jax.numpy as jnp
from jax import lax
from jax.experimental import pallas as pl
from jax.experimental.pallas import tpu as pltpu
from jax.experimental.pallas import tpu_sc as plsc

F_IN = 128
N_HEADS = 4
N_NODES = 10000
N_EDGES = 320000

NUM_CORES = 2
NUM_SUBCORES = 16
NUM_WORKERS = NUM_CORES * NUM_SUBCORES
LANES = 16

TW = 2 * N_HEADS
BLK_E = 128
BLK_W = BLK_E * N_HEADS
N_BLOCKS = N_EDGES // BLK_E
BPW = N_BLOCKS // NUM_WORKERS
TAIL_BLOCKS = N_BLOCKS - BPW * NUM_WORKERS
SUBB = 39
N_SUB = BPW // SUBB
SUB_E = SUBB * BLK_E
GROUPS = SUBB * (BLK_E // LANES)


def _node_projection(x, w):
    def body(x_ref, w_ref, y_ref):
        y_ref[...] = jnp.dot(x_ref[...], w_ref[...],
                             preferred_element_type=jnp.float32)

    m_blk = 2000
    return pl.pallas_call(
        body,
        grid=(N_NODES // m_blk,),
        in_specs=[pl.BlockSpec((m_blk, F_IN), lambda i: (i, 0)),
                  pl.BlockSpec((F_IN, TW), lambda i: (0, 0))],
        out_specs=pl.BlockSpec((m_blk, TW), lambda i: (i, 0)),
        out_shape=jax.ShapeDtypeStruct((N_NODES, TW), jnp.float32),
    )(x, w)


def _make_edge_lift():
    mesh = plsc.VectorSubcoreMesh(core_axis_name="c", subcore_axis_name="s")

    @functools.partial(
        pl.kernel,
        out_type=jax.ShapeDtypeStruct((N_EDGES * N_HEADS,), jnp.float32),
        mesh=mesh,
        compiler_params=pltpu.CompilerParams(needs_layout_passes=False),
        scratch_types=[
            pltpu.VMEM((N_NODES * TW,), jnp.float32),
            pltpu.VMEM((SUB_E,), jnp.int32),
            pltpu.VMEM((SUB_E,), jnp.int32),
            pltpu.VMEM((SUBB * BLK_W,), jnp.float32),
        ],
    )
    def edge_lift(y_hbm, src_hbm, dst_hbm, out_hbm,
                  table_v, src_v, dst_v, out_v):
        wid = lax.axis_index("s") * NUM_CORES + lax.axis_index("c")
        pltpu.sync_copy(y_hbm, table_v)

        def run_chunk(eb, wb, n_groups):
            pltpu.sync_copy(src_hbm.at[pl.ds(eb, n_groups * LANES)],
                            src_v.at[pl.ds(0, n_groups * LANES)])
            pltpu.sync_copy(dst_hbm.at[pl.ds(eb, n_groups * LANES)],
                            dst_v.at[pl.ds(0, n_groups * LANES)])

            def group(g, carry):
                s = src_v[pl.ds(g * LANES, LANES)]
                d = dst_v[pl.ds(g * LANES, LANES)]
                s8 = lax.shift_left(s, 3)
                d8 = lax.shift_left(d, 3) + N_HEADS
                base = lax.shift_left(lax.shift_right_logical(g, 3), 9) \
                    + lax.shift_left(lax.bitwise_and(g, 7), 4)
                for k in range(N_HEADS):
                    a = plsc.load_gather(table_v, [s8 + k])
                    b = plsc.load_gather(table_v, [d8 + k])
                    v = jnp.maximum(a + b, 0.0)
                    out_v[pl.ds(base + k * BLK_E, LANES)] = v
                return carry

            lax.fori_loop(0, n_groups, group, 0)
            pltpu.sync_copy(out_v.at[pl.ds(0, n_groups * LANES * N_HEADS)],
                            out_hbm.at[pl.ds(wb * BLK_W,
                                             n_groups * LANES * N_HEADS)])

        base_blk = wid * BPW
        for c in range(N_SUB):
            wb = base_blk + c * SUBB
            run_chunk(wb * BLK_E, wb, GROUPS)

        @pl.when(wid == NUM_WORKERS - 1)
        def _tail():
            wb = N_BLOCKS - TAIL_BLOCKS
            run_chunk(wb * BLK_E, wb, TAIL_BLOCKS * (BLK_E // LANES))

    return edge_lift


def kernel(x, edge_index, att):
    att2 = att[:, :, 0]
    w = jnp.concatenate([att2[:, :F_IN].T, att2[:, F_IN:].T], axis=1)
    y = _node_projection(x, w)
    out_flat = _make_edge_lift()(y.reshape(-1), edge_index[0], edge_index[1])
    return (out_flat.reshape(N_BLOCKS, N_HEADS, BLK_E)
            .transpose(0, 2, 1)
            .reshape(N_EDGES, N_HEADS))


# speedup vs baseline: 24.1059x; 1.2598x over previous
import functools

import jax
import jax.numpy as jnp
from jax import lax
from jax.experimental import pallas as pl
from jax.experimental.pallas import tpu as pltpu
from jax.experimental.pallas import tpu_sc as plsc

F_IN = 128
N_HEADS = 4
N_NODES = 10000
N_EDGES = 320000

NUM_CORES = 2
NUM_SUBCORES = 16
NUM_WORKERS = NUM_CORES * NUM_SUBCORES
LANES = 16

TW = 2 * N_HEADS
BLK_E = 128
BLK_W = BLK_E * N_HEADS
N_BLOCKS = N_EDGES // BLK_E
BPW = N_BLOCKS // NUM_WORKERS
TAIL_BLOCKS = N_BLOCKS - BPW * NUM_WORKERS
SUBB = 39
N_SUB = BPW // SUBB
SUB_E = SUBB * BLK_E
GROUPS = SUBB * (BLK_E // LANES)


def _node_projection(x, w):
    def body(x_ref, w_ref, y_ref):
        y_ref[...] = jnp.dot(x_ref[...], w_ref[...],
                             preferred_element_type=jnp.float32)

    m_blk = 2000
    return pl.pallas_call(
        body,
        grid=(N_NODES // m_blk,),
        in_specs=[pl.BlockSpec((m_blk, F_IN), lambda i: (i, 0)),
                  pl.BlockSpec((F_IN, TW), lambda i: (0, 0))],
        out_specs=pl.BlockSpec((m_blk, TW), lambda i: (i, 0)),
        out_shape=jax.ShapeDtypeStruct((N_NODES, TW), jnp.float32),
    )(x, w)


def _make_edge_lift():
    mesh = plsc.VectorSubcoreMesh(core_axis_name="c", subcore_axis_name="s")

    @functools.partial(
        pl.kernel,
        out_type=jax.ShapeDtypeStruct((N_EDGES * N_HEADS,), jnp.float32),
        mesh=mesh,
        compiler_params=pltpu.CompilerParams(needs_layout_passes=False),
        scratch_types=[
            pltpu.VMEM((N_NODES * TW,), jnp.float32),
            pltpu.VMEM((SUB_E,), jnp.int32),
            pltpu.VMEM((SUB_E,), jnp.int32),
            pltpu.VMEM((SUBB * BLK_W,), jnp.float32),
        ],
    )
    def edge_lift(y_hbm, src_hbm, dst_hbm, out_hbm,
                  table_v, src_v, dst_v, out_v):
        wid = lax.axis_index("s") * NUM_CORES + lax.axis_index("c")
        pltpu.sync_copy(y_hbm, table_v)

        def run_chunk(eb, wb, n_groups):
            pltpu.sync_copy(src_hbm.at[pl.ds(eb, n_groups * LANES)],
                            src_v.at[pl.ds(0, n_groups * LANES)])
            pltpu.sync_copy(dst_hbm.at[pl.ds(eb, n_groups * LANES)],
                            dst_v.at[pl.ds(0, n_groups * LANES)])

            @plsc.parallel_loop(0, n_groups, unroll=8)
            def group(g):
                s = src_v[pl.ds(g * LANES, LANES)]
                d = dst_v[pl.ds(g * LANES, LANES)]
                s8 = lax.shift_left(s, 3)
                d8 = lax.shift_left(d, 3) + N_HEADS
                base = lax.shift_left(lax.shift_right_logical(g, 3), 9) \
                    + lax.shift_left(lax.bitwise_and(g, 7), 4)
                for k in range(N_HEADS):
                    a = plsc.load_gather(table_v, [s8 + k])
                    b = plsc.load_gather(table_v, [d8 + k])
                    v = jnp.maximum(a + b, 0.0)
                    out_v[pl.ds(base + k * BLK_E, LANES)] = v
            pltpu.sync_copy(out_v.at[pl.ds(0, n_groups * LANES * N_HEADS)],
                            out_hbm.at[pl.ds(wb * BLK_W,
                                             n_groups * LANES * N_HEADS)])

        base_blk = wid * BPW
        for c in range(N_SUB):
            wb = base_blk + c * SUBB
            run_chunk(wb * BLK_E, wb, GROUPS)

        @pl.when(wid == NUM_WORKERS - 1)
        def _tail():
            wb = N_BLOCKS - TAIL_BLOCKS
            run_chunk(wb * BLK_E, wb, TAIL_BLOCKS * (BLK_E // LANES))

    return edge_lift


def kernel(x, edge_index, att):
    att2 = att[:, :, 0]
    w = jnp.concatenate([att2[:, :F_IN].T, att2[:, F_IN:].T], axis=1)
    y = _node_projection(x, w)
    out_flat = _make_edge_lift()(y.reshape(-1), edge_index[0], edge_index[1])
    return (out_flat.reshape(N_BLOCKS, N_HEADS, BLK_E)
            .transpose(0, 2, 1)
            .reshape(N_EDGES, N_HEADS))


# speedup vs baseline: 36.0267x; 1.4945x over previous
import functools

import jax
import jax.numpy as jnp
from jax import lax
from jax.experimental import pallas as pl
from jax.experimental.pallas import tpu as pltpu
from jax.experimental.pallas import tpu_sc as plsc

F_IN = 128
N_HEADS = 4
N_NODES = 10000
N_EDGES = 320000

NUM_CORES = 2
NUM_SUBCORES = 16
NUM_WORKERS = NUM_CORES * NUM_SUBCORES
LANES = 16

TW = 2 * N_HEADS
BLK_E = 128
NODE_BLOCKS = 80
TBL_W = NODE_BLOCKS * TW * BLK_E
N_BLOCKS = N_EDGES // BLK_E
BLK_W = BLK_E * N_HEADS
EI_W = 2 * BLK_E
BPW = N_BLOCKS // NUM_WORKERS
TAIL_BLOCKS = N_BLOCKS - BPW * NUM_WORKERS
SUBB = 39
N_SUB = BPW // SUBB
GROUPS = SUBB * (BLK_E // LANES)


def _node_projection(x, a):
    def body(x_ref, a_ref, y_ref):
        for b in range(8):
            xb = x_ref[pl.ds(b * BLK_E, BLK_E), :]
            y_ref[b] = lax.dot_general(a_ref[...], xb,
                                       (((1,), (1,)), ((), ())),
                                       preferred_element_type=jnp.float32)

    m_blk = 8 * BLK_E
    return pl.pallas_call(
        body,
        grid=(NODE_BLOCKS // 8,),
        in_specs=[pl.BlockSpec((m_blk, F_IN), lambda i: (i, 0)),
                  pl.BlockSpec((TW, F_IN), lambda i: (0, 0))],
        out_specs=pl.BlockSpec((8, TW, BLK_E), lambda i: (i, 0, 0)),
        out_shape=jax.ShapeDtypeStruct((NODE_BLOCKS, TW, BLK_E), jnp.float32),
    )(x, a)


def _make_edge_lift():
    mesh = plsc.VectorSubcoreMesh(core_axis_name="c", subcore_axis_name="s")

    @functools.partial(
        pl.kernel,
        out_type=jax.ShapeDtypeStruct((N_EDGES * N_HEADS,), jnp.float32),
        mesh=mesh,
        compiler_params=pltpu.CompilerParams(needs_layout_passes=False),
        scratch_types=[
            pltpu.VMEM((TBL_W,), jnp.float32),
            pltpu.VMEM((SUBB * BLK_E,), jnp.int32),
            pltpu.VMEM((SUBB * BLK_E,), jnp.int32),
            pltpu.VMEM((SUBB * BLK_W,), jnp.float32),
        ],
    )
    def edge_lift(y_hbm, ei_hbm, out_hbm, table_v, src_v, dst_v, out_v):
        wid = lax.axis_index("s") * NUM_CORES + lax.axis_index("c")
        pltpu.sync_copy(y_hbm, table_v)

        def run_chunk(wb, n_blocks):
            ne = n_blocks * BLK_E
            pltpu.sync_copy(ei_hbm.at[0, pl.ds(wb * BLK_E, ne)],
                            src_v.at[pl.ds(0, ne)])
            pltpu.sync_copy(ei_hbm.at[1, pl.ds(wb * BLK_E, ne)],
                            dst_v.at[pl.ds(0, ne)])

            @plsc.parallel_loop(0, n_blocks * (BLK_E // LANES), unroll=8)
            def group(g):
                blk = lax.shift_right_logical(g, 3)
                off = lax.shift_left(lax.bitwise_and(g, 7), 4)
                s = src_v[pl.ds(g * LANES, LANES)]
                d = dst_v[pl.ds(g * LANES, LANES)]
                sb = lax.shift_left(lax.shift_right_logical(s, 7), 10) \
                    + lax.bitwise_and(s, BLK_E - 1)
                db = lax.shift_left(lax.shift_right_logical(d, 7), 10) \
                    + lax.bitwise_and(d, BLK_E - 1) + N_HEADS * BLK_E
                base = lax.shift_left(blk, 9) + off
                for k in range(N_HEADS):
                    a = plsc.load_gather(table_v, [sb + k * BLK_E])
                    b = plsc.load_gather(table_v, [db + k * BLK_E])
                    v = jnp.maximum(a + b, 0.0)
                    out_v[pl.ds(base + k * BLK_E, LANES)] = v

            pltpu.sync_copy(out_v.at[pl.ds(0, n_blocks * BLK_W)],
                            out_hbm.at[pl.ds(wb * BLK_W, n_blocks * BLK_W)])

        base_blk = wid * BPW
        for c in range(N_SUB):
            run_chunk(base_blk + c * SUBB, SUBB)

        @pl.when(wid == NUM_WORKERS - 1)
        def _tail():
            run_chunk(N_BLOCKS - TAIL_BLOCKS, TAIL_BLOCKS)

    return edge_lift


def kernel(x, edge_index, att):
    att2 = att[:, :, 0]
    a = jnp.concatenate([att2[:, :F_IN], att2[:, F_IN:]], axis=0)
    y3 = _node_projection(x, a)
    out_flat = _make_edge_lift()(y3.reshape(-1), edge_index)
    return (out_flat.reshape(N_BLOCKS, N_HEADS, BLK_E)
            .transpose(0, 2, 1)
            .reshape(N_EDGES, N_HEADS))


# speedup vs baseline: 36.7806x; 1.0209x over previous
import functools

import jax
import jax.numpy as jnp
from jax import lax
from jax.experimental import pallas as pl
from jax.experimental.pallas import tpu as pltpu
from jax.experimental.pallas import tpu_sc as plsc

F_IN = 128
N_HEADS = 4
N_NODES = 10000
N_EDGES = 320000

NUM_CORES = 2
NUM_SUBCORES = 16
NUM_WORKERS = NUM_CORES * NUM_SUBCORES
LANES = 16

TW = 2 * N_HEADS
BLK_E = 128
NODE_BLOCKS = 80
NPAD = NODE_BLOCKS * BLK_E
TBL_W = TW * NPAD
N_BLOCKS = N_EDGES // BLK_E
BLK_W = BLK_E * N_HEADS
EI_W = 2 * BLK_E
BPW = N_BLOCKS // NUM_WORKERS
TAIL_BLOCKS = N_BLOCKS - BPW * NUM_WORKERS
SUBB = 39
N_SUB = BPW // SUBB
GROUPS = SUBB * (BLK_E // LANES)


def _node_projection(x, a):
    def body(x_ref, a_ref, y_ref):
        yt = lax.dot_general(a_ref[...], x_ref[...],
                             (((1,), (1,)), ((), ())),
                             preferred_element_type=jnp.float32)
        y_ref[...] = yt.reshape(TW, 8, BLK_E)

    m_blk = 8 * BLK_E
    return pl.pallas_call(
        body,
        grid=(NODE_BLOCKS // 8,),
        in_specs=[pl.BlockSpec((m_blk, F_IN), lambda i: (i, 0)),
                  pl.BlockSpec((TW, F_IN), lambda i: (0, 0))],
        out_specs=pl.BlockSpec((TW, 8, BLK_E), lambda i: (0, i, 0)),
        out_shape=jax.ShapeDtypeStruct((TW, NODE_BLOCKS, BLK_E), jnp.float32),
    )(x, a)


def _make_edge_lift():
    mesh = plsc.VectorSubcoreMesh(core_axis_name="c", subcore_axis_name="s")

    @functools.partial(
        pl.kernel,
        out_type=jax.ShapeDtypeStruct((N_EDGES * N_HEADS,), jnp.float32),
        mesh=mesh,
        compiler_params=pltpu.CompilerParams(needs_layout_passes=False),
        scratch_types=[
            pltpu.VMEM((TBL_W,), jnp.float32),
            pltpu.VMEM((SUBB * BLK_E,), jnp.int32),
            pltpu.VMEM((SUBB * BLK_E,), jnp.int32),
            pltpu.VMEM((SUBB * BLK_W,), jnp.float32),
        ],
    )
    def edge_lift(y_hbm, ei_hbm, out_hbm, table_v, src_v, dst_v, out_v):
        wid = lax.axis_index("s") * NUM_CORES + lax.axis_index("c")
        pltpu.sync_copy(y_hbm, table_v)

        def run_chunk(wb, n_blocks):
            ne = n_blocks * BLK_E
            pltpu.sync_copy(ei_hbm.at[0, pl.ds(wb * BLK_E, ne)],
                            src_v.at[pl.ds(0, ne)])
            pltpu.sync_copy(ei_hbm.at[1, pl.ds(wb * BLK_E, ne)],
                            dst_v.at[pl.ds(0, ne)])

            @plsc.parallel_loop(0, n_blocks * (BLK_E // LANES), unroll=8)
            def group(g):
                blk = lax.shift_right_logical(g, 3)
                off = lax.shift_left(lax.bitwise_and(g, 7), 4)
                s = src_v[pl.ds(g * LANES, LANES)]
                d = dst_v[pl.ds(g * LANES, LANES)]
                base = lax.shift_left(blk, 9) + off
                for k in range(N_HEADS):
                    a = plsc.load_gather(
                        table_v.at[pl.ds(k * NPAD, NPAD)], [s])
                    b = plsc.load_gather(
                        table_v.at[pl.ds((N_HEADS + k) * NPAD, NPAD)], [d])
                    v = jnp.maximum(a + b, 0.0)
                    out_v[pl.ds(base + k * BLK_E, LANES)] = v

            pltpu.sync_copy(out_v.at[pl.ds(0, n_blocks * BLK_W)],
                            out_hbm.at[pl.ds(wb * BLK_W, n_blocks * BLK_W)])

        base_blk = wid * BPW
        for c in range(N_SUB):
            run_chunk(base_blk + c * SUBB, SUBB)

        @pl.when(wid == NUM_WORKERS - 1)
        def _tail():
            run_chunk(N_BLOCKS - TAIL_BLOCKS, TAIL_BLOCKS)

    return edge_lift


def kernel(x, edge_index, att):
    att2 = att[:, :, 0]
    a = jnp.concatenate([att2[:, :F_IN], att2[:, F_IN:]], axis=0)
    y3 = _node_projection(x, a)
    out_flat = _make_edge_lift()(y3.reshape(-1), edge_index)
    return (out_flat.reshape(N_BLOCKS, N_HEADS, BLK_E)
            .transpose(0, 2, 1)
            .reshape(N_EDGES, N_HEADS))


# speedup vs baseline: 37.0672x; 1.0078x over previous
import functools

import jax
import jax.numpy as jnp
from jax import lax
from jax.experimental import pallas as pl
from jax.experimental.pallas import tpu as pltpu
from jax.experimental.pallas import tpu_sc as plsc

F_IN = 128
N_HEADS = 4
N_NODES = 10000
N_EDGES = 320000

NUM_CORES = 2
NUM_SUBCORES = 16
NUM_WORKERS = NUM_CORES * NUM_SUBCORES
LANES = 16

TW = 2 * N_HEADS
BLK_E = 128
NODE_BLOCKS = 80
NPAD = NODE_BLOCKS * BLK_E
TBL_W = TW * NPAD
N_BLOCKS = N_EDGES // BLK_E
BLK_W = BLK_E * N_HEADS
EI_W = 2 * BLK_E
BPW = N_BLOCKS // NUM_WORKERS
TAIL_BLOCKS = N_BLOCKS - BPW * NUM_WORKERS
SUBB = 39
N_SUB = BPW // SUBB
GROUPS = SUBB * (BLK_E // LANES)


def _node_projection(x, a):
    def body(x_ref, a_ref, y_ref):
        yt = lax.dot_general(a_ref[...], x_ref[...],
                             (((1,), (1,)), ((), ())),
                             preferred_element_type=jnp.float32)
        y_ref[...] = yt.reshape(TW, 8, BLK_E)

    m_blk = 8 * BLK_E
    return pl.pallas_call(
        body,
        grid=(NODE_BLOCKS // 8,),
        in_specs=[pl.BlockSpec((m_blk, F_IN), lambda i: (i, 0)),
                  pl.BlockSpec((TW, F_IN), lambda i: (0, 0))],
        out_specs=pl.BlockSpec((TW, 8, BLK_E), lambda i: (0, i, 0)),
        out_shape=jax.ShapeDtypeStruct((TW, NODE_BLOCKS, BLK_E), jnp.float32),
    )(x, a)


def _make_edge_lift():
    mesh = plsc.VectorSubcoreMesh(core_axis_name="c", subcore_axis_name="s")

    @functools.partial(
        pl.kernel,
        out_type=jax.ShapeDtypeStruct((N_EDGES * N_HEADS,), jnp.float32),
        mesh=mesh,
        compiler_params=pltpu.CompilerParams(needs_layout_passes=False),
        scratch_types=[
            pltpu.VMEM((TBL_W,), jnp.float32),
            pltpu.VMEM((SUBB * BLK_E,), jnp.int32),
            pltpu.VMEM((SUBB * BLK_E,), jnp.int32),
            pltpu.VMEM((SUBB * BLK_W,), jnp.float32),
        ],
    )
    def edge_lift(y_hbm, ei_hbm, out_hbm, table_v, src_v, dst_v, out_v):
        wid = lax.axis_index("s") * NUM_CORES + lax.axis_index("c")
        pltpu.sync_copy(y_hbm, table_v)

        def run_chunk(wb, n_blocks):
            ne = n_blocks * BLK_E
            pltpu.sync_copy(ei_hbm.at[0, pl.ds(wb * BLK_E, ne)],
                            src_v.at[pl.ds(0, ne)])
            pltpu.sync_copy(ei_hbm.at[1, pl.ds(wb * BLK_E, ne)],
                            dst_v.at[pl.ds(0, ne)])

            @plsc.parallel_loop(0, n_blocks * (BLK_E // LANES), unroll=8)
            def group(g):
                blk = lax.shift_right_logical(g, 3)
                off = lax.shift_left(lax.bitwise_and(g, 7), 4)
                s = src_v[pl.ds(g * LANES, LANES)]
                d = dst_v[pl.ds(g * LANES, LANES)]
                base = lax.shift_left(blk, 9) + off
                for k in range(N_HEADS):
                    a = plsc.load_gather(
                        table_v.at[pl.ds(k * NPAD, NPAD)], [s])
                    b = plsc.load_gather(
                        table_v.at[pl.ds((N_HEADS + k) * NPAD, NPAD)], [d])
                    v = jnp.maximum(a + b, 0.0)
                    out_v[pl.ds(base + k * BLK_E, LANES)] = v

            pltpu.sync_copy(out_v.at[pl.ds(0, n_blocks * BLK_W)],
                            out_hbm.at[pl.ds(wb * BLK_W, n_blocks * BLK_W)])

        base_blk = wid * BPW

        def chunk_body(c, carry):
            run_chunk(base_blk + c * SUBB, SUBB)
            return carry

        lax.fori_loop(0, N_SUB, chunk_body, 0)

        @pl.when(wid == NUM_WORKERS - 1)
        def _tail():
            run_chunk(N_BLOCKS - TAIL_BLOCKS, TAIL_BLOCKS)

    return edge_lift


def kernel(x, edge_index, att):
    att2 = att[:, :, 0]
    a = jnp.concatenate([att2[:, :F_IN], att2[:, F_IN:]], axis=0)
    y3 = _node_projection(x, a)
    out_flat = _make_edge_lift()(y3.reshape(-1), edge_index)
    return (out_flat.reshape(N_BLOCKS, N_HEADS, BLK_E)
            .transpose(0, 2, 1)
            .reshape(N_EDGES, N_HEADS))


# speedup vs baseline: 44.2571x; 1.1940x over previous
import functools

import jax
import jax.numpy as jnp
from jax import lax
from jax.experimental import pallas as pl
from jax.experimental.pallas import tpu as pltpu
from jax.experimental.pallas import tpu_sc as plsc

F_IN = 128
N_HEADS = 4
N_NODES = 10000
N_EDGES = 320000

NUM_CORES = 2
NUM_SUBCORES = 16
NUM_WORKERS = NUM_CORES * NUM_SUBCORES
LANES = 16

TW = 2 * N_HEADS
PW = N_HEADS
BLK_E = 128
NODE_BLOCKS = 80
NPAD = NODE_BLOCKS * BLK_E
TBL_W = PW * NPAD
N_BLOCKS = N_EDGES // BLK_E
BLK_W = BLK_E * N_HEADS
BPW = N_BLOCKS // NUM_WORKERS
TAIL_BLOCKS = N_BLOCKS - BPW * NUM_WORKERS
HI_MASK = -65536


def _node_projection(x, a):
    def body(x_ref, a_ref, y_ref):
        yt = lax.dot_general(a_ref[...], x_ref[...],
                             (((1,), (1,)), ((), ())),
                             preferred_element_type=jnp.float32)
        u = lax.bitcast_convert_type(yt.astype(jnp.bfloat16),
                                     jnp.uint16).astype(jnp.uint32)
        packed = u[:PW] | (u[PW:] << 16)
        y_ref[...] = lax.bitcast_convert_type(
            packed, jnp.int32).reshape(PW, 8, BLK_E)

    m_blk = 8 * BLK_E
    return pl.pallas_call(
        body,
        grid=(NODE_BLOCKS // 8,),
        in_specs=[pl.BlockSpec((m_blk, F_IN), lambda i: (i, 0)),
                  pl.BlockSpec((TW, F_IN), lambda i: (0, 0))],
        out_specs=pl.BlockSpec((PW, 8, BLK_E), lambda i: (0, i, 0)),
        out_shape=jax.ShapeDtypeStruct((PW, NODE_BLOCKS, BLK_E), jnp.int32),
    )(x, a)


def _make_edge_lift():
    mesh = plsc.VectorSubcoreMesh(core_axis_name="c", subcore_axis_name="s")

    @functools.partial(
        pl.kernel,
        out_type=jax.ShapeDtypeStruct((N_EDGES * N_HEADS,), jnp.float32),
        mesh=mesh,
        compiler_params=pltpu.CompilerParams(needs_layout_passes=False),
        scratch_types=[
            pltpu.VMEM((TBL_W,), jnp.int32),
            pltpu.VMEM((BPW * BLK_E,), jnp.int32),
            pltpu.VMEM((BPW * BLK_E,), jnp.int32),
            pltpu.VMEM((BPW * BLK_W,), jnp.float32),
        ],
    )
    def edge_lift(y_hbm, ei_hbm, out_hbm, table_v, src_v, dst_v, out_v):
        wid = lax.axis_index("s") * NUM_CORES + lax.axis_index("c")
        pltpu.sync_copy(y_hbm, table_v)

        def unpack(w):
            lo = plsc.bitcast(lax.shift_left(w, 16), jnp.float32)
            hi = plsc.bitcast(lax.bitwise_and(w, HI_MASK), jnp.float32)
            return lo, hi

        def run_chunk(wb, n_blocks):
            ne = n_blocks * BLK_E
            pltpu.sync_copy(ei_hbm.at[0, pl.ds(wb * BLK_E, ne)],
                            src_v.at[pl.ds(0, ne)])
            pltpu.sync_copy(ei_hbm.at[1, pl.ds(wb * BLK_E, ne)],
                            dst_v.at[pl.ds(0, ne)])

            @plsc.parallel_loop(0, n_blocks * (BLK_E // LANES), unroll=8)
            def group(g):
                blk = lax.shift_right_logical(g, 3)
                off = lax.shift_left(lax.bitwise_and(g, 7), 4)
                s = src_v[pl.ds(g * LANES, LANES)]
                d = dst_v[pl.ds(g * LANES, LANES)]
                base = lax.shift_left(blk, 9) + off
                s01, s23, d01, d23 = (
                    plsc.load_gather(table_v.at[pl.ds(p * NPAD, NPAD)], [i])
                    for p, i in ((0, s), (1, s), (2, d), (3, d)))
                h0, h1 = unpack(s01)
                h2, h3 = unpack(s23)
                g0, g1 = unpack(d01)
                g2, g3 = unpack(d23)
                for k, (hv, gv) in enumerate(
                        ((h0, g0), (h1, g1), (h2, g2), (h3, g3))):
                    out_v[pl.ds(base + k * BLK_E, LANES)] = \
                        jnp.maximum(hv + gv, 0.0)

            pltpu.sync_copy(out_v.at[pl.ds(0, n_blocks * BLK_W)],
                            out_hbm.at[pl.ds(wb * BLK_W, n_blocks * BLK_W)])

        run_chunk(wid * BPW, BPW)

        @pl.when(wid == NUM_WORKERS - 1)
        def _tail():
            run_chunk(N_BLOCKS - TAIL_BLOCKS, TAIL_BLOCKS)

    return edge_lift


def kernel(x, edge_index, att):
    att2 = att[:, :, 0]
    att_s = att2[:, :F_IN]
    att_d = att2[:, F_IN:]
    a = jnp.concatenate([att_s[0::2], att_d[0::2],
                         att_s[1::2], att_d[1::2]], axis=0)
    y3 = _node_projection(x, a)
    out_flat = _make_edge_lift()(y3.reshape(-1), edge_index)
    return (out_flat.reshape(N_BLOCKS, N_HEADS, BLK_E)
            .transpose(0, 2, 1)
            .reshape(N_EDGES, N_HEADS))


# speedup vs baseline: 46.1380x; 1.0425x over previous
import functools

import jax
import jax.numpy as jnp
from jax import lax
from jax.experimental import pallas as pl
from jax.experimental.pallas import tpu as pltpu
from jax.experimental.pallas import tpu_sc as plsc

F_IN = 128
N_HEADS = 4
N_NODES = 10000
N_EDGES = 320000

NUM_CORES = 2
NUM_SUBCORES = 16
NUM_WORKERS = NUM_CORES * NUM_SUBCORES
LANES = 16

TW = 2 * N_HEADS
PW = N_HEADS
BLK_E = 128
NODE_BLOCKS = 80
NPAD = NODE_BLOCKS * BLK_E
TBL_W = PW * NPAD
N_BLOCKS = N_EDGES // BLK_E
BLK_W = BLK_E * N_HEADS
BPW = N_BLOCKS // NUM_WORKERS
TAIL_BLOCKS = N_BLOCKS - BPW * NUM_WORKERS
HI_MASK = -65536


def _node_projection(x, a):
    def body(x_ref, a_ref, y_ref):
        yt = lax.dot_general(a_ref[...], x_ref[...],
                             (((1,), (1,)), ((), ())),
                             preferred_element_type=jnp.float32)
        u = lax.bitcast_convert_type(yt.astype(jnp.bfloat16),
                                     jnp.uint16).astype(jnp.uint32)
        packed = u[:PW] | (u[PW:] << 16)
        y_ref[...] = lax.bitcast_convert_type(
            packed, jnp.int32).reshape(PW, -1, BLK_E)

    nb_step = 16
    m_blk = nb_step * BLK_E
    return pl.pallas_call(
        body,
        grid=(NODE_BLOCKS // nb_step,),
        in_specs=[pl.BlockSpec((m_blk, F_IN), lambda i: (i, 0)),
                  pl.BlockSpec((TW, F_IN), lambda i: (0, 0))],
        out_specs=pl.BlockSpec((PW, nb_step, BLK_E), lambda i: (0, i, 0)),
        out_shape=jax.ShapeDtypeStruct((PW, NODE_BLOCKS, BLK_E), jnp.int32),
    )(x, a)


def _make_edge_lift():
    mesh = plsc.VectorSubcoreMesh(core_axis_name="c", subcore_axis_name="s")

    @functools.partial(
        pl.kernel,
        out_type=jax.ShapeDtypeStruct((N_EDGES * N_HEADS,), jnp.float32),
        mesh=mesh,
        compiler_params=pltpu.CompilerParams(needs_layout_passes=False),
        scratch_types=[
            pltpu.VMEM((TBL_W,), jnp.int32),
            pltpu.VMEM((BPW * BLK_E,), jnp.int32),
            pltpu.VMEM((BPW * BLK_E,), jnp.int32),
            pltpu.VMEM((BPW * BLK_W,), jnp.float32),
            pltpu.SemaphoreType.DMA,
        ],
    )
    def edge_lift(y_hbm, ei_hbm, out_hbm, table_v, src_v, dst_v, out_v, sem):
        wid = lax.axis_index("s") * NUM_CORES + lax.axis_index("c")
        pltpu.sync_copy(y_hbm, table_v)

        def unpack(w):
            lo = plsc.bitcast(lax.shift_left(w, 16), jnp.float32)
            hi = plsc.bitcast(lax.bitwise_and(w, HI_MASK), jnp.float32)
            return lo, hi

        def load_ids(wb, n_blocks):
            ne = n_blocks * BLK_E
            pltpu.sync_copy(ei_hbm.at[0, pl.ds(wb * BLK_E, ne)],
                            src_v.at[pl.ds(0, ne)])
            pltpu.sync_copy(ei_hbm.at[1, pl.ds(wb * BLK_E, ne)],
                            dst_v.at[pl.ds(0, ne)])

        def gather_groups(g_lo, g_hi):
            @plsc.parallel_loop(g_lo, g_hi, unroll=16)
            def group(g):
                blk = lax.shift_right_logical(g, 3)
                off = lax.shift_left(lax.bitwise_and(g, 7), 4)
                s = src_v[pl.ds(g * LANES, LANES)]
                d = dst_v[pl.ds(g * LANES, LANES)]
                base = lax.shift_left(blk, 9) + off
                s01, s23, d01, d23 = (
                    plsc.load_gather(table_v.at[pl.ds(p * NPAD, NPAD)], [i])
                    for p, i in ((0, s), (1, s), (2, d), (3, d)))
                h0, h1 = unpack(s01)
                h2, h3 = unpack(s23)
                g0, g1 = unpack(d01)
                g2, g3 = unpack(d23)
                for k, (hv, gv) in enumerate(
                        ((h0, g0), (h1, g1), (h2, g2), (h3, g3))):
                    out_v[pl.ds(base + k * BLK_E, LANES)] = \
                        jnp.maximum(hv + gv, 0.0)

        def store_out(b_lo, b_hi, wb):
            nw = (b_hi - b_lo) * BLK_W
            return pltpu.async_copy(
                out_v.at[pl.ds(b_lo * BLK_W, nw)],
                out_hbm.at[pl.ds((wb + b_lo) * BLK_W, nw)], sem)

        wb = wid * BPW
        half = 40
        load_ids(wb, BPW)
        gather_groups(0, half * (BLK_E // LANES))
        c1 = store_out(0, half, wb)
        gather_groups(half * (BLK_E // LANES), BPW * (BLK_E // LANES))
        c2 = store_out(half, BPW, wb)
        c1.wait()
        c2.wait()

        @pl.when(wid == NUM_WORKERS - 1)
        def _tail():
            twb = N_BLOCKS - TAIL_BLOCKS
            load_ids(twb, TAIL_BLOCKS)
            gather_groups(0, TAIL_BLOCKS * (BLK_E // LANES))
            store_out(0, TAIL_BLOCKS, twb).wait()

    return edge_lift


def kernel(x, edge_index, att):
    att2 = att[:, :, 0]
    att_s = att2[:, :F_IN]
    att_d = att2[:, F_IN:]
    a = jnp.concatenate([att_s[0::2], att_d[0::2],
                         att_s[1::2], att_d[1::2]], axis=0)
    y3 = _node_projection(x, a)
    out_flat = _make_edge_lift()(y3.reshape(-1), edge_index)
    return (out_flat.reshape(N_BLOCKS, N_HEADS, BLK_E)
            .transpose(0, 2, 1)
            .reshape(N_EDGES, N_HEADS))


# speedup vs baseline: 47.2506x; 1.0241x over previous
import functools

import jax
import jax.numpy as jnp
from jax import lax
from jax.experimental import pallas as pl
from jax.experimental.pallas import tpu as pltpu
from jax.experimental.pallas import tpu_sc as plsc

F_IN = 128
N_HEADS = 4
N_NODES = 10000
N_EDGES = 320000

NUM_CORES = 2
NUM_SUBCORES = 16
NUM_WORKERS = NUM_CORES * NUM_SUBCORES
LANES = 16

TW = 2 * N_HEADS
PW = N_HEADS
BLK_E = 128
NODE_BLOCKS = 80
NPAD = NODE_BLOCKS * BLK_E
TBL_W = PW * NPAD
N_BLOCKS = N_EDGES // BLK_E
BLK_W = BLK_E * N_HEADS
BPW = 80
HI_MASK = -65536


def _node_projection(x, a):
    def body(x_ref, a_ref, y_ref):
        yt = lax.dot_general(a_ref[...], x_ref[...],
                             (((1,), (1,)), ((), ())),
                             preferred_element_type=jnp.float32)
        u = lax.bitcast_convert_type(yt.astype(jnp.bfloat16),
                                     jnp.uint16).astype(jnp.uint32)
        packed = u[:PW] | (u[PW:] << 16)
        y_ref[...] = lax.bitcast_convert_type(
            packed, jnp.int32).reshape(PW, -1, BLK_E)

    nb_step = 16
    m_blk = nb_step * BLK_E
    return pl.pallas_call(
        body,
        grid=(NODE_BLOCKS // nb_step,),
        in_specs=[pl.BlockSpec((m_blk, F_IN), lambda i: (i, 0)),
                  pl.BlockSpec((TW, F_IN), lambda i: (0, 0))],
        out_specs=pl.BlockSpec((PW, nb_step, BLK_E), lambda i: (0, i, 0)),
        out_shape=jax.ShapeDtypeStruct((PW, NODE_BLOCKS, BLK_E), jnp.int32),
    )(x, a)


def _make_edge_lift():
    mesh = plsc.VectorSubcoreMesh(core_axis_name="c", subcore_axis_name="s")

    @functools.partial(
        pl.kernel,
        out_type=jax.ShapeDtypeStruct((N_EDGES * N_HEADS,), jnp.float32),
        mesh=mesh,
        compiler_params=pltpu.CompilerParams(needs_layout_passes=False),
        scratch_types=[
            pltpu.VMEM((TBL_W,), jnp.int32),
            pltpu.VMEM((BPW * BLK_E,), jnp.int32),
            pltpu.VMEM((BPW * BLK_E,), jnp.int32),
            pltpu.VMEM((BPW * BLK_W,), jnp.float32),
            pltpu.SemaphoreType.DMA,
        ],
    )
    def edge_lift(y_hbm, ei_hbm, out_hbm, table_v, src_v, dst_v, out_v, sem):
        wid = lax.axis_index("s") * NUM_CORES + lax.axis_index("c")
        pltpu.sync_copy(y_hbm, table_v)

        def unpack(w):
            lo = plsc.bitcast(lax.shift_left(w, 16), jnp.float32)
            hi = plsc.bitcast(lax.bitwise_and(w, HI_MASK), jnp.float32)
            return lo, hi

        def load_ids(wb, n_blocks):
            ne = n_blocks * BLK_E
            pltpu.sync_copy(ei_hbm.at[0, pl.ds(wb * BLK_E, ne)],
                            src_v.at[pl.ds(0, ne)])
            pltpu.sync_copy(ei_hbm.at[1, pl.ds(wb * BLK_E, ne)],
                            dst_v.at[pl.ds(0, ne)])

        def gather_groups(g_lo, g_hi):
            @plsc.parallel_loop(g_lo, g_hi, unroll=16)
            def group(g):
                blk = lax.shift_right_logical(g, 3)
                off = lax.shift_left(lax.bitwise_and(g, 7), 4)
                s = src_v[pl.ds(g * LANES, LANES)]
                d = dst_v[pl.ds(g * LANES, LANES)]
                base = lax.shift_left(blk, 9) + off
                s01, s23, d01, d23 = (
                    plsc.load_gather(table_v.at[pl.ds(p * NPAD, NPAD)], [i])
                    for p, i in ((0, s), (1, s), (2, d), (3, d)))
                h0, h1 = unpack(s01)
                h2, h3 = unpack(s23)
                g0, g1 = unpack(d01)
                g2, g3 = unpack(d23)
                for k, (hv, gv) in enumerate(
                        ((h0, g0), (h1, g1), (h2, g2), (h3, g3))):
                    out_v[pl.ds(base + k * BLK_E, LANES)] = \
                        jnp.maximum(hv + gv, 0.0)

        def store_out(b_lo, b_hi, wb):
            nw = (b_hi - b_lo) * BLK_W
            return pltpu.async_copy(
                out_v.at[pl.ds(b_lo * BLK_W, nw)],
                out_hbm.at[pl.ds((wb + b_lo) * BLK_W, nw)], sem)

        wb = lax.div(wid * (N_BLOCKS - BPW), NUM_WORKERS - 1)
        half = BPW // 2
        load_ids(wb, BPW)
        gather_groups(0, half * (BLK_E // LANES))
        c1 = store_out(0, half, wb)
        gather_groups(half * (BLK_E // LANES), BPW * (BLK_E // LANES))
        c2 = store_out(half, BPW, wb)
        c1.wait()
        c2.wait()

    return edge_lift


def kernel(x, edge_index, att):
    att2 = att[:, :, 0]
    att_s = att2[:, :F_IN]
    att_d = att2[:, F_IN:]
    a = jnp.concatenate([att_s[0::2], att_d[0::2],
                         att_s[1::2], att_d[1::2]], axis=0)
    y3 = _node_projection(x, a)
    out_flat = _make_edge_lift()(y3.reshape(-1), edge_index)
    return (out_flat.reshape(N_BLOCKS, N_HEADS, BLK_E)
            .transpose(0, 2, 1)
            .reshape(N_EDGES, N_HEADS))
